# baseline (speedup 1.0000x reference)
import functools

import jax
import jax.numpy as jnp
from jax import lax
from jax.experimental import pallas as pl
from jax.experimental.pallas import tpu as pltpu
from jax.experimental.pallas import tpu_sc as plsc

NC = 2
NS = 16
NW = NC * NS
EB = 128
JB = 4
BB = 8
DD = 16
RZ = 128


def _pad_sizes(n, e):
    rt = -(-(-(-e // EB) // NW) // (2 * JB)) * (2 * JB)
    e_pad = rt * NW * EB
    npt = -(-(n // NS) // RZ) * RZ
    n_acc = npt * NS
    return e_pad, n_acc, npt, rt


def _make_mesh():
    return plsc.VectorSubcoreMesh(core_axis_name="c", subcore_axis_name="s")


def _make_agg(n, e, d):
    assert n % NS == 0 and d % 16 == 0
    e_pad, n_acc, npt, rt = _pad_sizes(n, e)
    nblocks = e_pad // (BB * EB)

    @functools.partial(
        pl.kernel,
        mesh=_make_mesh(),
        out_type=jax.ShapeDtypeStruct((NC, n_acc, d), jnp.float32),
        scratch_types=[
            pltpu.VMEM((BB, EB), jnp.int32),
            pltpu.VMEM((BB, EB), jnp.int32),
            pltpu.VMEM((EB, d), jnp.float32),
            pltpu.VMEM((EB, d), jnp.float32),
            pltpu.VMEM_SHARED((n_acc, d), jnp.float32),
            pltpu.SemaphoreType.DMA,
        ],
        compiler_params=pltpu.CompilerParams(use_tc_tiling_on_sc=False),
    )
    def agg(g_hbm, src_hbm, dst_hbm, out_hbm, idx_s, idx_d, rows0, rows1,
            acc, gsem):
        rows = [rows0, rows1]
        c = lax.axis_index("c")
        s = lax.axis_index("s")
        wid = s * NC + c

        def zrow(i, carry):
            for k in range(d // 16):
                rows0[i, pl.ds(k * 16, 16)] = jnp.zeros((16,), jnp.float32)
            return carry

        lax.fori_loop(0, EB, zrow, 0)
        for i in range(npt // RZ):
            pltpu.sync_copy(rows0, acc.at[pl.ds(s * npt + i * RZ, RZ)])
        plsc.subcore_barrier()

        niter = (nblocks - wid + NW - 1) // NW

        def ebody(t, carry):
            off = (wid + t * NW) * BB
            pltpu.sync_copy(src_hbm.at[pl.ds(off, BB)], idx_s)
            pltpu.sync_copy(dst_hbm.at[pl.ds(off, BB)], idx_d)
            for j in range(BB):
                pltpu.async_copy(g_hbm.at[idx_s.at[j]], rows0, gsem).wait()
                pltpu.sync_copy(rows0, acc.at[idx_d.at[j]], add=True)
            return carry

        lax.fori_loop(0, niter, ebody, 0)
        plsc.subcore_barrier()

        for i in range(npt // RZ):
            r0 = s * npt + i * RZ
            pltpu.sync_copy(acc.at[pl.ds(r0, RZ)],
                            out_hbm.at[c].at[pl.ds(r0, RZ)])

    return agg


def _make_deg(n, e):
    assert n % NS == 0
    e_pad, n_acc, npt, rt = _pad_sizes(n, e)

    @functools.partial(
        pl.kernel,
        mesh=_make_mesh(),
        out_type=jax.ShapeDtypeStruct((NC, n_acc, DD), jnp.float32),
        scratch_types=[
            pltpu.VMEM((rt, EB), jnp.int32),
            pltpu.VMEM((EB, DD), jnp.float32),
            pltpu.VMEM_SHARED((n_acc, DD), jnp.float32),
            pltpu.SemaphoreType.DMA,
            pltpu.SemaphoreType.DMA,
        ],
        compiler_params=pltpu.CompilerParams(use_tc_tiling_on_sc=False),
    )
    def deg(dst_hbm, out_hbm, idx_d, buf, acc, gsem, ssem):
        c = lax.axis_index("c")
        s = lax.axis_index("s")
        wid = s * NC + c

        pltpu.async_copy(dst_hbm.at[pl.ds(wid * rt, rt)], idx_d, gsem)

        def fill(v):
            def frow(i, carry):
                buf[i, pl.ds(0, 16)] = jnp.full((16,), v, jnp.float32)
                return carry
            lax.fori_loop(0, EB, frow, 0)

        fill(0.0)
        zdesc = []
        for i in range(npt // RZ):
            zdesc.append(pltpu.async_copy(
                buf.at[pl.ds(0, RZ)],
                acc.at[pl.ds(s * npt + i * RZ, RZ)], ssem))
        for dsc in zdesc:
            dsc.wait()
        plsc.subcore_barrier()
        fill(1.0)
        pltpu.make_async_copy(dst_hbm.at[pl.ds(wid * rt, rt)], idx_d,
                              gsem).wait()

        batch = 2 * JB

        def ebody(t, carry):
            sdesc = [pltpu.async_copy(buf, acc.at[idx_d.at[t * batch + j]],
                                      ssem, add=True)
                     for j in range(batch)]
            for dsc in sdesc:
                dsc.wait()
            return carry

        lax.fori_loop(0, rt // batch, ebody, 0)
        plsc.subcore_barrier()

        odesc = []
        for i in range(npt // RZ):
            r0 = s * npt + i * RZ
            odesc.append(pltpu.async_copy(
                acc.at[pl.ds(r0, RZ)], out_hbm.at[c].at[pl.ds(r0, RZ)],
                gsem))
        for dsc in odesc:
            dsc.wait()

    return deg


def _dinv(dp, n):
    deg = dp[0, :n, 0:1] + dp[1, :n, 0:1] + 1.0
    return lax.rsqrt(deg)


def _tc_first(x, w1, dparts):
    n, d_out = x.shape[0], w1.shape[1]

    def body(x_ref, w_ref, dp_ref, h_ref, g_ref):
        dinv = _dinv(dp_ref[...], n)
        h = jnp.dot(x_ref[...], w_ref[...], preferred_element_type=jnp.float32)
        h_ref[...] = h
        g_ref[...] = h * dinv

    return pl.pallas_call(
        body,
        out_shape=(jax.ShapeDtypeStruct((n, d_out), jnp.float32),
                   jax.ShapeDtypeStruct((n, d_out), jnp.float32)),
    )(x, w1, dparts)


def _tc_mid(h_prev, parts, w, b_prev, dparts):
    n, d_out = h_prev.shape[0], w.shape[1]

    def body(h_ref, p_ref, w_ref, b_ref, dp_ref, h2_ref, g2_ref):
        dinv = _dinv(dp_ref[...], n)
        p = p_ref[...]
        agg = p[0, :n] + p[1, :n]
        a = jnp.maximum(dinv * agg + (dinv * dinv) * h_ref[...] + b_ref[...],
                        0.0)
        h2 = jnp.dot(a, w_ref[...], preferred_element_type=jnp.float32)
        h2_ref[...] = h2
        g2_ref[...] = h2 * dinv

    return pl.pallas_call(
        body,
        out_shape=(jax.ShapeDtypeStruct((n, d_out), jnp.float32),
                   jax.ShapeDtypeStruct((n, d_out), jnp.float32)),
    )(h_prev, parts, w, b_prev, dparts)


def _tc_final(h_prev, parts, b_prev, dparts):
    n, d = h_prev.shape

    def body(h_ref, p_ref, b_ref, dp_ref, o_ref):
        dinv = _dinv(dp_ref[...], n)
        p = p_ref[...]
        agg = p[0, :n] + p[1, :n]
        a = jnp.maximum(dinv * agg + (dinv * dinv) * h_ref[...] + b_ref[...],
                        0.0)
        z = a - jnp.max(a, axis=1, keepdims=True)
        lse = jnp.log(jnp.sum(jnp.exp(z), axis=1, keepdims=True))
        o_ref[...] = z - lse

    return pl.pallas_call(
        body,
        out_shape=jax.ShapeDtypeStruct((n, d), jnp.float32),
    )(h_prev, parts, b_prev, dparts)


def kernel(x, edge_index, W1, b1, W2, b2, W3, b3):
    n = x.shape[0]
    e = edge_index.shape[1]
    e_pad, n_acc, _, _ = _pad_sizes(n, e)
    ei = edge_index.astype(jnp.int32)
    pad_dst = n + jnp.arange(e_pad - e, dtype=jnp.int32) % (n_acc - n)
    src2 = jnp.concatenate(
        [ei[0], jnp.zeros((e_pad - e,), jnp.int32)]).reshape(e_pad // EB, EB)
    dst2 = jnp.concatenate(
        [ei[1], pad_dst]).reshape(e_pad // EB, EB)

    dparts = _make_deg(n, e)(dst2)
    h1, g1 = _tc_first(x, W1, dparts)
    p1 = _make_agg(n, e, h1.shape[1])(g1, src2, dst2)
    h2, g2 = _tc_mid(h1, p1, W2, b1.reshape(1, -1), dparts)
    p2 = _make_agg(n, e, h2.shape[1])(g2, src2, dst2)
    h3, g3 = _tc_mid(h2, p2, W3, b2.reshape(1, -1), dparts)
    p3 = _make_agg(n, e, h3.shape[1])(g3, src2, dst2)
    return _tc_final(h3, p3, b3.reshape(1, -1), dparts)


# speedup vs baseline: 1.8919x; 1.8919x over previous
import functools

import jax
import jax.numpy as jnp
from jax import lax
from jax.experimental import pallas as pl
from jax.experimental.pallas import tpu as pltpu
from jax.experimental.pallas import tpu_sc as plsc

NC = 2
NS = 16
NW = NC * NS
EB = 128
JB = 4
BB = 8
DD = 16
RZ = 128


def _pad_sizes(n, e):
    rt = -(-(-(-e // EB) // NW) // (2 * JB)) * (2 * JB)
    e_pad = rt * NW * EB
    npt = -(-(n // NS) // RZ) * RZ
    n_acc = npt * NS
    return e_pad, n_acc, npt, rt


def _make_mesh():
    return plsc.VectorSubcoreMesh(core_axis_name="c", subcore_axis_name="s")


def _make_agg(n, e, d):
    assert n % NS == 0 and d % 16 == 0
    _, n_acc, npt, _ = _pad_sizes(n, e)
    e_pad = -(-e // (BB * EB)) * (BB * EB)
    nblocks = e_pad // (BB * EB)

    @functools.partial(
        pl.kernel,
        mesh=_make_mesh(),
        out_type=jax.ShapeDtypeStruct((NC, n_acc, d), jnp.float32),
        scratch_types=[
            pltpu.VMEM((BB, EB), jnp.int32),
            pltpu.VMEM((BB, EB), jnp.int32),
            pltpu.VMEM((EB, d), jnp.float32),
            pltpu.VMEM((EB, d), jnp.float32),
            pltpu.VMEM_SHARED((n_acc, d), jnp.float32),
            pltpu.SemaphoreType.DMA,
        ],
        compiler_params=pltpu.CompilerParams(use_tc_tiling_on_sc=False),
    )
    def agg(g_hbm, src_hbm, dst_hbm, out_hbm, idx_s, idx_d, rows0, rows1,
            acc, gsem):
        rows = [rows0, rows1]
        c = lax.axis_index("c")
        s = lax.axis_index("s")
        wid = s * NC + c

        def zrow(i, carry):
            for k in range(d // 16):
                rows0[i, pl.ds(k * 16, 16)] = jnp.zeros((16,), jnp.float32)
            return carry

        lax.fori_loop(0, EB, zrow, 0)
        for i in range(npt // RZ):
            pltpu.sync_copy(rows0, acc.at[pl.ds(s * npt + i * RZ, RZ)])
        plsc.subcore_barrier()

        niter = (nblocks - wid + NW - 1) // NW

        def ebody(t, carry):
            off = (wid + t * NW) * BB
            pltpu.sync_copy(src_hbm.at[pl.ds(off, BB)], idx_s)
            pltpu.sync_copy(dst_hbm.at[pl.ds(off, BB)], idx_d)
            g = pltpu.async_copy(g_hbm.at[idx_s.at[0]], rows[0], gsem)
            for j in range(BB):
                if j + 1 < BB:
                    gn = pltpu.async_copy(g_hbm.at[idx_s.at[j + 1]],
                                          rows[(j + 1) % 2], gsem)
                g.wait()
                pltpu.sync_copy(rows[j % 2], acc.at[idx_d.at[j]],
                                add=True)
                if j + 1 < BB:
                    g = gn
            return carry

        lax.fori_loop(0, niter, ebody, 0)
        plsc.subcore_barrier()

        for i in range(npt // RZ):
            r0 = s * npt + i * RZ
            pltpu.sync_copy(acc.at[pl.ds(r0, RZ)],
                            out_hbm.at[c].at[pl.ds(r0, RZ)])

    return agg


def _make_deg(n, e):
    assert n % NS == 0
    e_pad, n_acc, npt, rt = _pad_sizes(n, e)

    @functools.partial(
        pl.kernel,
        mesh=_make_mesh(),
        out_type=jax.ShapeDtypeStruct((NC, n_acc, DD), jnp.float32),
        scratch_types=[
            pltpu.VMEM((rt, EB), jnp.int32),
            pltpu.VMEM((EB, DD), jnp.float32),
            pltpu.VMEM_SHARED((n_acc, DD), jnp.float32),
            pltpu.SemaphoreType.DMA,
            pltpu.SemaphoreType.DMA,
        ],
        compiler_params=pltpu.CompilerParams(use_tc_tiling_on_sc=False),
    )
    def deg(dst_hbm, out_hbm, idx_d, buf, acc, gsem, ssem):
        c = lax.axis_index("c")
        s = lax.axis_index("s")
        wid = s * NC + c

        pltpu.async_copy(dst_hbm.at[pl.ds(wid * rt, rt)], idx_d, gsem)

        def fill(v):
            def frow(i, carry):
                buf[i, pl.ds(0, 16)] = jnp.full((16,), v, jnp.float32)
                return carry
            lax.fori_loop(0, EB, frow, 0)

        fill(0.0)
        zdesc = []
        for i in range(npt // RZ):
            zdesc.append(pltpu.async_copy(
                buf.at[pl.ds(0, RZ)],
                acc.at[pl.ds(s * npt + i * RZ, RZ)], ssem))
        for dsc in zdesc:
            dsc.wait()
        plsc.subcore_barrier()
        fill(1.0)
        pltpu.make_async_copy(dst_hbm.at[pl.ds(wid * rt, rt)], idx_d,
                              gsem).wait()

        batch = 2 * JB

        def ebody(t, carry):
            sdesc = [pltpu.async_copy(buf, acc.at[idx_d.at[t * batch + j]],
                                      ssem, add=True)
                     for j in range(batch)]
            for dsc in sdesc:
                dsc.wait()
            return carry

        lax.fori_loop(0, rt // batch, ebody, 0)
        plsc.subcore_barrier()

        odesc = []
        for i in range(npt // RZ):
            r0 = s * npt + i * RZ
            odesc.append(pltpu.async_copy(
                acc.at[pl.ds(r0, RZ)], out_hbm.at[c].at[pl.ds(r0, RZ)],
                gsem))
        for dsc in odesc:
            dsc.wait()

    return deg


def _dinv(dp, n):
    deg = dp[0, :n, 0:1] + dp[1, :n, 0:1] + 1.0
    return lax.rsqrt(deg)


def _tc_first(x, w1, dparts):
    n, d_out = x.shape[0], w1.shape[1]

    def body(x_ref, w_ref, dp_ref, h_ref, g_ref):
        dinv = _dinv(dp_ref[...], n)
        h = jnp.dot(x_ref[...], w_ref[...], preferred_element_type=jnp.float32)
        h_ref[...] = h
        g_ref[...] = h * dinv

    return pl.pallas_call(
        body,
        out_shape=(jax.ShapeDtypeStruct((n, d_out), jnp.float32),
                   jax.ShapeDtypeStruct((n, d_out), jnp.float32)),
    )(x, w1, dparts)


def _tc_mid(h_prev, parts, w, b_prev, dparts):
    n, d_out = h_prev.shape[0], w.shape[1]

    def body(h_ref, p_ref, w_ref, b_ref, dp_ref, h2_ref, g2_ref):
        dinv = _dinv(dp_ref[...], n)
        p = p_ref[...]
        agg = p[0, :n] + p[1, :n]
        a = jnp.maximum(dinv * agg + (dinv * dinv) * h_ref[...] + b_ref[...],
                        0.0)
        h2 = jnp.dot(a, w_ref[...], preferred_element_type=jnp.float32)
        h2_ref[...] = h2
        g2_ref[...] = h2 * dinv

    return pl.pallas_call(
        body,
        out_shape=(jax.ShapeDtypeStruct((n, d_out), jnp.float32),
                   jax.ShapeDtypeStruct((n, d_out), jnp.float32)),
    )(h_prev, parts, w, b_prev, dparts)


def _tc_final(h_prev, parts, b_prev, dparts):
    n, d = h_prev.shape

    def body(h_ref, p_ref, b_ref, dp_ref, o_ref):
        dinv = _dinv(dp_ref[...], n)
        p = p_ref[...]
        agg = p[0, :n] + p[1, :n]
        a = jnp.maximum(dinv * agg + (dinv * dinv) * h_ref[...] + b_ref[...],
                        0.0)
        z = a - jnp.max(a, axis=1, keepdims=True)
        lse = jnp.log(jnp.sum(jnp.exp(z), axis=1, keepdims=True))
        o_ref[...] = z - lse

    return pl.pallas_call(
        body,
        out_shape=jax.ShapeDtypeStruct((n, d), jnp.float32),
    )(h_prev, parts, b_prev, dparts)


def kernel(x, edge_index, W1, b1, W2, b2, W3, b3):
    n = x.shape[0]
    e = edge_index.shape[1]
    e_pad_d, n_acc, _, _ = _pad_sizes(n, e)
    e_pad = -(-e // (BB * EB)) * (BB * EB)
    ei = edge_index.astype(jnp.int32)
    pad_dst = n + jnp.arange(e_pad_d - e, dtype=jnp.int32) % (n_acc - n)
    src2 = jnp.concatenate(
        [ei[0], jnp.zeros((e_pad - e,), jnp.int32)]).reshape(e_pad // EB, EB)
    dst2 = jnp.concatenate(
        [ei[1], pad_dst[:e_pad - e]]).reshape(e_pad // EB, EB)
    dst2d = jnp.concatenate(
        [ei[1], pad_dst]).reshape(e_pad_d // EB, EB)

    dparts = _make_deg(n, e)(dst2d)
    h1, g1 = _tc_first(x, W1, dparts)
    p1 = _make_agg(n, e, h1.shape[1])(g1, src2, dst2)
    h2, g2 = _tc_mid(h1, p1, W2, b1.reshape(1, -1), dparts)
    p2 = _make_agg(n, e, h2.shape[1])(g2, src2, dst2)
    h3, g3 = _tc_mid(h2, p2, W3, b2.reshape(1, -1), dparts)
    p3 = _make_agg(n, e, h3.shape[1])(g3, src2, dst2)
    return _tc_final(h3, p3, b3.reshape(1, -1), dparts)


# speedup vs baseline: 2.1534x; 1.1382x over previous
import functools

import jax
import jax.numpy as jnp
from jax import lax
from jax.experimental import pallas as pl
from jax.experimental.pallas import tpu as pltpu
from jax.experimental.pallas import tpu_sc as plsc

NC = 2
NS = 16
NW = NC * NS
EB = 128
JB = 4
BB = 8
DD = 16
RZ = 128


def _pad_sizes(n, e):
    rt = -(-(-(-e // EB) // NW) // (2 * JB)) * (2 * JB)
    e_pad = rt * NW * EB
    npt = -(-(n // NS) // RZ) * RZ
    n_acc = npt * NS
    return e_pad, n_acc, npt, rt


def _make_mesh():
    return plsc.VectorSubcoreMesh(core_axis_name="c", subcore_axis_name="s")


def _make_agg(n, e, d):
    assert n % NS == 0 and d % 16 == 0
    e_pad, n_acc, npt, rt = _pad_sizes(n, e)

    @functools.partial(
        pl.kernel,
        mesh=_make_mesh(),
        out_type=jax.ShapeDtypeStruct((NC, n_acc, d), jnp.float32),
        scratch_types=[
            pltpu.VMEM((rt, EB), jnp.int32),
            pltpu.VMEM((rt, EB), jnp.int32),
            pltpu.VMEM((EB, d), jnp.float32),
            pltpu.VMEM((EB, d), jnp.float32),
            pltpu.VMEM_SHARED((n_acc, d), jnp.float32),
            pltpu.SemaphoreType.DMA,
            pltpu.SemaphoreType.DMA,
        ],
        compiler_params=pltpu.CompilerParams(use_tc_tiling_on_sc=False),
    )
    def agg(g_hbm, src_hbm, dst_hbm, out_hbm, idx_s, idx_d, rows0, rows1,
            acc, gsem, isem):
        rows = [rows0, rows1]
        c = lax.axis_index("c")
        s = lax.axis_index("s")
        wid = s * NC + c

        pltpu.async_copy(src_hbm.at[pl.ds(wid * rt, rt)], idx_s, isem)
        pltpu.async_copy(dst_hbm.at[pl.ds(wid * rt, rt)], idx_d, isem)

        def zrow(i, carry):
            for k in range(d // 16):
                rows0[i, pl.ds(k * 16, 16)] = jnp.zeros((16,), jnp.float32)
            return carry

        lax.fori_loop(0, EB, zrow, 0)
        for i in range(npt // RZ):
            pltpu.sync_copy(rows0, acc.at[pl.ds(s * npt + i * RZ, RZ)])
        pltpu.make_async_copy(src_hbm.at[pl.ds(wid * rt, rt)], idx_s,
                              isem).wait()
        pltpu.make_async_copy(dst_hbm.at[pl.ds(wid * rt, rt)], idx_d,
                              isem).wait()
        plsc.subcore_barrier()

        def ebody(t, carry):
            g = pltpu.async_copy(g_hbm.at[idx_s.at[t * BB]], rows[0], gsem)
            for j in range(BB):
                if j + 1 < BB:
                    gn = pltpu.async_copy(g_hbm.at[idx_s.at[t * BB + j + 1]],
                                          rows[(j + 1) % 2], gsem)
                g.wait()
                pltpu.sync_copy(rows[j % 2], acc.at[idx_d.at[t * BB + j]],
                                add=True)
                if j + 1 < BB:
                    g = gn
            return carry

        lax.fori_loop(0, rt // BB, ebody, 0)
        plsc.subcore_barrier()

        for i in range(npt // RZ):
            r0 = s * npt + i * RZ
            pltpu.sync_copy(acc.at[pl.ds(r0, RZ)],
                            out_hbm.at[c].at[pl.ds(r0, RZ)])

    return agg


def _make_deg(n, e):
    assert n % NS == 0
    e_pad, n_acc, npt, rt = _pad_sizes(n, e)

    @functools.partial(
        pl.kernel,
        mesh=_make_mesh(),
        out_type=jax.ShapeDtypeStruct((NC, n_acc, DD), jnp.float32),
        scratch_types=[
            pltpu.VMEM((rt, EB), jnp.int32),
            pltpu.VMEM((EB, DD), jnp.float32),
            pltpu.VMEM_SHARED((n_acc, DD), jnp.float32),
            pltpu.SemaphoreType.DMA,
            pltpu.SemaphoreType.DMA,
        ],
        compiler_params=pltpu.CompilerParams(use_tc_tiling_on_sc=False),
    )
    def deg(dst_hbm, out_hbm, idx_d, buf, acc, gsem, ssem):
        c = lax.axis_index("c")
        s = lax.axis_index("s")
        wid = s * NC + c

        pltpu.async_copy(dst_hbm.at[pl.ds(wid * rt, rt)], idx_d, gsem)

        def fill(v):
            def frow(i, carry):
                buf[i, pl.ds(0, 16)] = jnp.full((16,), v, jnp.float32)
                return carry
            lax.fori_loop(0, EB, frow, 0)

        fill(0.0)
        zdesc = []
        for i in range(npt // RZ):
            zdesc.append(pltpu.async_copy(
                buf.at[pl.ds(0, RZ)],
                acc.at[pl.ds(s * npt + i * RZ, RZ)], ssem))
        for dsc in zdesc:
            dsc.wait()
        plsc.subcore_barrier()
        fill(1.0)
        pltpu.make_async_copy(dst_hbm.at[pl.ds(wid * rt, rt)], idx_d,
                              gsem).wait()

        batch = 2 * JB

        def ebody(t, carry):
            sdesc = [pltpu.async_copy(buf, acc.at[idx_d.at[t * batch + j]],
                                      ssem, add=True)
                     for j in range(batch)]
            for dsc in sdesc:
                dsc.wait()
            return carry

        lax.fori_loop(0, rt // batch, ebody, 0)
        plsc.subcore_barrier()

        odesc = []
        for i in range(npt // RZ):
            r0 = s * npt + i * RZ
            odesc.append(pltpu.async_copy(
                acc.at[pl.ds(r0, RZ)], out_hbm.at[c].at[pl.ds(r0, RZ)],
                gsem))
        for dsc in odesc:
            dsc.wait()

    return deg


def _dinv(dp, n):
    deg = dp[0, :n, 0:1] + dp[1, :n, 0:1] + 1.0
    return lax.rsqrt(deg)


def _tc_first(x, w1, dparts):
    n, d_out = x.shape[0], w1.shape[1]

    def body(x_ref, w_ref, dp_ref, h_ref, g_ref):
        dinv = _dinv(dp_ref[...], n)
        h = jnp.dot(x_ref[...], w_ref[...], preferred_element_type=jnp.float32)
        h_ref[...] = h
        g_ref[...] = h * dinv

    return pl.pallas_call(
        body,
        out_shape=(jax.ShapeDtypeStruct((n, d_out), jnp.float32),
                   jax.ShapeDtypeStruct((n, d_out), jnp.float32)),
    )(x, w1, dparts)


def _tc_mid(h_prev, parts, w, b_prev, dparts):
    n, d_out = h_prev.shape[0], w.shape[1]

    def body(h_ref, p_ref, w_ref, b_ref, dp_ref, h2_ref, g2_ref):
        dinv = _dinv(dp_ref[...], n)
        p = p_ref[...]
        agg = p[0, :n] + p[1, :n]
        a = jnp.maximum(dinv * agg + (dinv * dinv) * h_ref[...] + b_ref[...],
                        0.0)
        h2 = jnp.dot(a, w_ref[...], preferred_element_type=jnp.float32)
        h2_ref[...] = h2
        g2_ref[...] = h2 * dinv

    return pl.pallas_call(
        body,
        out_shape=(jax.ShapeDtypeStruct((n, d_out), jnp.float32),
                   jax.ShapeDtypeStruct((n, d_out), jnp.float32)),
    )(h_prev, parts, w, b_prev, dparts)


def _tc_final(h_prev, parts, b_prev, dparts):
    n, d = h_prev.shape

    def body(h_ref, p_ref, b_ref, dp_ref, o_ref):
        dinv = _dinv(dp_ref[...], n)
        p = p_ref[...]
        agg = p[0, :n] + p[1, :n]
        a = jnp.maximum(dinv * agg + (dinv * dinv) * h_ref[...] + b_ref[...],
                        0.0)
        z = a - jnp.max(a, axis=1, keepdims=True)
        lse = jnp.log(jnp.sum(jnp.exp(z), axis=1, keepdims=True))
        o_ref[...] = z - lse

    return pl.pallas_call(
        body,
        out_shape=jax.ShapeDtypeStruct((n, d), jnp.float32),
    )(h_prev, parts, b_prev, dparts)


def kernel(x, edge_index, W1, b1, W2, b2, W3, b3):
    n = x.shape[0]
    e = edge_index.shape[1]
    e_pad, n_acc, _, _ = _pad_sizes(n, e)
    ei = edge_index.astype(jnp.int32)
    npad = e_pad - e
    pad_src = jnp.arange(npad, dtype=jnp.int32) % n
    pad_dst = n + jnp.arange(npad, dtype=jnp.int32) % (n_acc - n)
    src2 = jnp.concatenate([ei[0], pad_src]).reshape(e_pad // EB, EB)
    dst2 = jnp.concatenate([ei[1], pad_dst]).reshape(e_pad // EB, EB)

    dparts = _make_deg(n, e)(dst2)
    h1, g1 = _tc_first(x, W1, dparts)
    p1 = _make_agg(n, e, h1.shape[1])(g1, src2, dst2)
    h2, g2 = _tc_mid(h1, p1, W2, b1.reshape(1, -1), dparts)
    p2 = _make_agg(n, e, h2.shape[1])(g2, src2, dst2)
    h3, g3 = _tc_mid(h2, p2, W3, b2.reshape(1, -1), dparts)
    p3 = _make_agg(n, e, h3.shape[1])(g3, src2, dst2)
    return _tc_final(h3, p3, b3.reshape(1, -1), dparts)


# speedup vs baseline: 2.1858x; 1.0150x over previous
import functools

import jax
import jax.numpy as jnp
from jax import lax
from jax.experimental import pallas as pl
from jax.experimental.pallas import tpu as pltpu
from jax.experimental.pallas import tpu_sc as plsc

NC = 2
NS = 16
NW = NC * NS
EB = 128
JB = 4
BB = 8
DD = 16
RZ = 128


def _pad_sizes(n, e):
    rt = -(-(-(-e // EB) // NW) // (2 * JB)) * (2 * JB)
    e_pad = rt * NW * EB
    npt = -(-(n // NS) // RZ) * RZ
    n_acc = npt * NS
    return e_pad, n_acc, npt, rt


def _make_mesh():
    return plsc.VectorSubcoreMesh(core_axis_name="c", subcore_axis_name="s")


def _make_agg(n, e, d):
    assert n % NS == 0 and d % 16 == 0
    e_pad, n_acc, npt, rt = _pad_sizes(n, e)

    @functools.partial(
        pl.kernel,
        mesh=_make_mesh(),
        out_type=jax.ShapeDtypeStruct((NC, n_acc, d), jnp.float32),
        scratch_types=[
            pltpu.VMEM((rt, EB), jnp.int32),
            pltpu.VMEM((rt, EB), jnp.int32),
            pltpu.VMEM((EB, d), jnp.float32),
            pltpu.VMEM((EB, d), jnp.float32),
            pltpu.VMEM((EB, d), jnp.float32),
            pltpu.VMEM((EB, d), jnp.float32),
            pltpu.VMEM_SHARED((n_acc, d), jnp.float32),
            pltpu.SemaphoreType.DMA,
            pltpu.SemaphoreType.DMA,
            pltpu.SemaphoreType.DMA,
        ],
        compiler_params=pltpu.CompilerParams(use_tc_tiling_on_sc=False),
    )
    def agg(g_hbm, src_hbm, dst_hbm, out_hbm, idx_s, idx_d, rows0, rows1,
            rows2, rows3, acc, gsem, ssem, isem):
        rows = [rows0, rows1, rows2, rows3]
        c = lax.axis_index("c")
        s = lax.axis_index("s")
        wid = s * NC + c

        pltpu.async_copy(src_hbm.at[pl.ds(wid * rt, rt)], idx_s, isem)
        pltpu.async_copy(dst_hbm.at[pl.ds(wid * rt, rt)], idx_d, isem)

        def zrow(i, carry):
            for k in range(d // 16):
                rows0[i, pl.ds(k * 16, 16)] = jnp.zeros((16,), jnp.float32)
            return carry

        lax.fori_loop(0, EB, zrow, 0)
        for i in range(npt // RZ):
            pltpu.sync_copy(rows0, acc.at[pl.ds(s * npt + i * RZ, RZ)])
        pltpu.make_async_copy(src_hbm.at[pl.ds(wid * rt, rt)], idx_s,
                              isem).wait()
        pltpu.make_async_copy(dst_hbm.at[pl.ds(wid * rt, rt)], idx_d,
                              isem).wait()
        plsc.subcore_barrier()

        def ebody(t, carry):
            gd = [None] * BB
            sd = [None] * BB
            gd[0] = pltpu.async_copy(g_hbm.at[idx_s.at[t * BB]],
                                     rows[0], gsem)
            gd[1] = pltpu.async_copy(g_hbm.at[idx_s.at[t * BB + 1]],
                                     rows[1], gsem)
            for j in range(BB):
                gd[j].wait()
                sd[j] = pltpu.async_copy(rows[j % 4],
                                         acc.at[idx_d.at[t * BB + j]],
                                         ssem, add=True)
                if j + 2 < BB:
                    if j - 2 >= 0:
                        sd[j - 2].wait()
                    gd[j + 2] = pltpu.async_copy(
                        g_hbm.at[idx_s.at[t * BB + j + 2]],
                        rows[(j + 2) % 4], gsem)
            for j in range(max(0, BB - 4), BB):
                sd[j].wait()
            return carry

        lax.fori_loop(0, rt // BB, ebody, 0)
        plsc.subcore_barrier()

        for i in range(npt // RZ):
            r0 = s * npt + i * RZ
            pltpu.sync_copy(acc.at[pl.ds(r0, RZ)],
                            out_hbm.at[c].at[pl.ds(r0, RZ)])

    return agg


def _make_deg(n, e):
    assert n % NS == 0
    e_pad, n_acc, npt, rt = _pad_sizes(n, e)

    @functools.partial(
        pl.kernel,
        mesh=_make_mesh(),
        out_type=jax.ShapeDtypeStruct((NC, n_acc, DD), jnp.float32),
        scratch_types=[
            pltpu.VMEM((rt, EB), jnp.int32),
            pltpu.VMEM((EB, DD), jnp.float32),
            pltpu.VMEM_SHARED((n_acc, DD), jnp.float32),
            pltpu.SemaphoreType.DMA,
            pltpu.SemaphoreType.DMA,
        ],
        compiler_params=pltpu.CompilerParams(use_tc_tiling_on_sc=False),
    )
    def deg(dst_hbm, out_hbm, idx_d, buf, acc, gsem, ssem):
        c = lax.axis_index("c")
        s = lax.axis_index("s")
        wid = s * NC + c

        pltpu.async_copy(dst_hbm.at[pl.ds(wid * rt, rt)], idx_d, gsem)

        def fill(v):
            def frow(i, carry):
                buf[i, pl.ds(0, 16)] = jnp.full((16,), v, jnp.float32)
                return carry
            lax.fori_loop(0, EB, frow, 0)

        fill(0.0)
        zdesc = []
        for i in range(npt // RZ):
            zdesc.append(pltpu.async_copy(
                buf.at[pl.ds(0, RZ)],
                acc.at[pl.ds(s * npt + i * RZ, RZ)], ssem))
        for dsc in zdesc:
            dsc.wait()
        plsc.subcore_barrier()
        fill(1.0)
        pltpu.make_async_copy(dst_hbm.at[pl.ds(wid * rt, rt)], idx_d,
                              gsem).wait()

        batch = 2 * JB

        def ebody(t, carry):
            sdesc = [pltpu.async_copy(buf, acc.at[idx_d.at[t * batch + j]],
                                      ssem, add=True)
                     for j in range(batch)]
            for dsc in sdesc:
                dsc.wait()
            return carry

        lax.fori_loop(0, rt // batch, ebody, 0)
        plsc.subcore_barrier()

        odesc = []
        for i in range(npt // RZ):
            r0 = s * npt + i * RZ
            odesc.append(pltpu.async_copy(
                acc.at[pl.ds(r0, RZ)], out_hbm.at[c].at[pl.ds(r0, RZ)],
                gsem))
        for dsc in odesc:
            dsc.wait()

    return deg


def _dinv(dp, n):
    deg = dp[0, :n, 0:1] + dp[1, :n, 0:1] + 1.0
    return lax.rsqrt(deg)


def _tc_first(x, w1, dparts):
    n, d_out = x.shape[0], w1.shape[1]

    def body(x_ref, w_ref, dp_ref, h_ref, g_ref):
        dinv = _dinv(dp_ref[...], n)
        h = jnp.dot(x_ref[...], w_ref[...], preferred_element_type=jnp.float32)
        h_ref[...] = h
        g_ref[...] = h * dinv

    return pl.pallas_call(
        body,
        out_shape=(jax.ShapeDtypeStruct((n, d_out), jnp.float32),
                   jax.ShapeDtypeStruct((n, d_out), jnp.float32)),
    )(x, w1, dparts)


def _tc_mid(h_prev, parts, w, b_prev, dparts):
    n, d_out = h_prev.shape[0], w.shape[1]

    def body(h_ref, p_ref, w_ref, b_ref, dp_ref, h2_ref, g2_ref):
        dinv = _dinv(dp_ref[...], n)
        p = p_ref[...]
        agg = p[0, :n] + p[1, :n]
        a = jnp.maximum(dinv * agg + (dinv * dinv) * h_ref[...] + b_ref[...],
                        0.0)
        h2 = jnp.dot(a, w_ref[...], preferred_element_type=jnp.float32)
        h2_ref[...] = h2
        g2_ref[...] = h2 * dinv

    return pl.pallas_call(
        body,
        out_shape=(jax.ShapeDtypeStruct((n, d_out), jnp.float32),
                   jax.ShapeDtypeStruct((n, d_out), jnp.float32)),
    )(h_prev, parts, w, b_prev, dparts)


def _tc_final(h_prev, parts, b_prev, dparts):
    n, d = h_prev.shape

    def body(h_ref, p_ref, b_ref, dp_ref, o_ref):
        dinv = _dinv(dp_ref[...], n)
        p = p_ref[...]
        agg = p[0, :n] + p[1, :n]
        a = jnp.maximum(dinv * agg + (dinv * dinv) * h_ref[...] + b_ref[...],
                        0.0)
        z = a - jnp.max(a, axis=1, keepdims=True)
        lse = jnp.log(jnp.sum(jnp.exp(z), axis=1, keepdims=True))
        o_ref[...] = z - lse

    return pl.pallas_call(
        body,
        out_shape=jax.ShapeDtypeStruct((n, d), jnp.float32),
    )(h_prev, parts, b_prev, dparts)


def kernel(x, edge_index, W1, b1, W2, b2, W3, b3):
    n = x.shape[0]
    e = edge_index.shape[1]
    e_pad, n_acc, _, _ = _pad_sizes(n, e)
    ei = edge_index.astype(jnp.int32)
    npad = e_pad - e
    pad_src = jnp.arange(npad, dtype=jnp.int32) % n
    pad_dst = n + jnp.arange(npad, dtype=jnp.int32) % (n_acc - n)
    src2 = jnp.concatenate([ei[0], pad_src]).reshape(e_pad // EB, EB)
    dst2 = jnp.concatenate([ei[1], pad_dst]).reshape(e_pad // EB, EB)

    dparts = _make_deg(n, e)(dst2)
    h1, g1 = _tc_first(x, W1, dparts)
    p1 = _make_agg(n, e, h1.shape[1])(g1, src2, dst2)
    h2, g2 = _tc_mid(h1, p1, W2, b1.reshape(1, -1), dparts)
    p2 = _make_agg(n, e, h2.shape[1])(g2, src2, dst2)
    h3, g3 = _tc_mid(h2, p2, W3, b2.reshape(1, -1), dparts)
    p3 = _make_agg(n, e, h3.shape[1])(g3, src2, dst2)
    return _tc_final(h3, p3, b3.reshape(1, -1), dparts)


# speedup vs baseline: 2.1864x; 1.0003x over previous
import functools

import jax
import jax.numpy as jnp
from jax import lax
from jax.experimental import pallas as pl
from jax.experimental.pallas import tpu as pltpu
from jax.experimental.pallas import tpu_sc as plsc

NC = 2
NS = 16
NW = NC * NS
EB = 128
JB = 4
BB = 8
DD = 16
RZ = 128


def _pad_sizes(n, e):
    rt = -(-(-(-e // EB) // NW) // (2 * JB)) * (2 * JB)
    e_pad = rt * NW * EB
    npt = -(-(n // NS) // RZ) * RZ
    n_acc = npt * NS
    return e_pad, n_acc, npt, rt


def _make_mesh():
    return plsc.VectorSubcoreMesh(core_axis_name="c", subcore_axis_name="s")


def _make_agg(n, e, d):
    assert n % NS == 0 and d % 16 == 0
    e_pad, n_acc, npt, rt = _pad_sizes(n, e)

    @functools.partial(
        pl.kernel,
        mesh=_make_mesh(),
        out_type=jax.ShapeDtypeStruct((NC, n_acc, d), jnp.float32),
        scratch_types=[
            pltpu.VMEM((rt, EB), jnp.int32),
            pltpu.VMEM((rt, EB), jnp.int32),
            pltpu.VMEM((EB, d), jnp.float32),
            pltpu.VMEM((EB, d), jnp.float32),
            pltpu.VMEM((EB, d), jnp.float32),
            pltpu.VMEM((EB, d), jnp.float32),
            pltpu.VMEM_SHARED((n_acc, d), jnp.float32),
            pltpu.SemaphoreType.DMA,
            pltpu.SemaphoreType.DMA,
            pltpu.SemaphoreType.DMA,
        ],
        compiler_params=pltpu.CompilerParams(use_tc_tiling_on_sc=False, skip_device_barrier=True),
    )
    def agg(g_hbm, src_hbm, dst_hbm, out_hbm, idx_s, idx_d, rows0, rows1,
            rows2, rows3, acc, gsem, ssem, isem):
        rows = [rows0, rows1, rows2, rows3]
        c = lax.axis_index("c")
        s = lax.axis_index("s")
        wid = s * NC + c

        pltpu.async_copy(src_hbm.at[pl.ds(wid * rt, rt)], idx_s, isem)
        pltpu.async_copy(dst_hbm.at[pl.ds(wid * rt, rt)], idx_d, isem)

        def zrow(i, carry):
            for k in range(d // 16):
                rows0[i, pl.ds(k * 16, 16)] = jnp.zeros((16,), jnp.float32)
            return carry

        lax.fori_loop(0, EB, zrow, 0)
        for i in range(npt // RZ):
            pltpu.sync_copy(rows0, acc.at[pl.ds(s * npt + i * RZ, RZ)])
        pltpu.make_async_copy(src_hbm.at[pl.ds(wid * rt, rt)], idx_s,
                              isem).wait()
        pltpu.make_async_copy(dst_hbm.at[pl.ds(wid * rt, rt)], idx_d,
                              isem).wait()
        plsc.subcore_barrier()

        def ebody(t, carry):
            gd = [None] * BB
            sd = [None] * BB
            gd[0] = pltpu.async_copy(g_hbm.at[idx_s.at[t * BB]],
                                     rows[0], gsem)
            gd[1] = pltpu.async_copy(g_hbm.at[idx_s.at[t * BB + 1]],
                                     rows[1], gsem)
            for j in range(BB):
                gd[j].wait()
                sd[j] = pltpu.async_copy(rows[j % 4],
                                         acc.at[idx_d.at[t * BB + j]],
                                         ssem, add=True)
                if j + 2 < BB:
                    if j - 2 >= 0:
                        sd[j - 2].wait()
                    gd[j + 2] = pltpu.async_copy(
                        g_hbm.at[idx_s.at[t * BB + j + 2]],
                        rows[(j + 2) % 4], gsem)
            for j in range(max(0, BB - 4), BB):
                sd[j].wait()
            return carry

        lax.fori_loop(0, rt // BB, ebody, 0)
        plsc.subcore_barrier()

        for i in range(npt // RZ):
            r0 = s * npt + i * RZ
            pltpu.sync_copy(acc.at[pl.ds(r0, RZ)],
                            out_hbm.at[c].at[pl.ds(r0, RZ)])

    return agg


def _make_deg(n, e):
    assert n % NS == 0
    e_pad, n_acc, npt, rt = _pad_sizes(n, e)

    @functools.partial(
        pl.kernel,
        mesh=_make_mesh(),
        out_type=jax.ShapeDtypeStruct((NC, n_acc, DD), jnp.float32),
        scratch_types=[
            pltpu.VMEM((rt, EB), jnp.int32),
            pltpu.VMEM((EB, DD), jnp.float32),
            pltpu.VMEM_SHARED((n_acc, DD), jnp.float32),
            pltpu.SemaphoreType.DMA,
            pltpu.SemaphoreType.DMA,
        ],
        compiler_params=pltpu.CompilerParams(use_tc_tiling_on_sc=False, skip_device_barrier=True),
    )
    def deg(dst_hbm, out_hbm, idx_d, buf, acc, gsem, ssem):
        c = lax.axis_index("c")
        s = lax.axis_index("s")
        wid = s * NC + c

        pltpu.async_copy(dst_hbm.at[pl.ds(wid * rt, rt)], idx_d, gsem)

        def fill(v):
            def frow(i, carry):
                buf[i, pl.ds(0, 16)] = jnp.full((16,), v, jnp.float32)
                return carry
            lax.fori_loop(0, EB, frow, 0)

        fill(0.0)
        zdesc = []
        for i in range(npt // RZ):
            zdesc.append(pltpu.async_copy(
                buf.at[pl.ds(0, RZ)],
                acc.at[pl.ds(s * npt + i * RZ, RZ)], ssem))
        for dsc in zdesc:
            dsc.wait()
        plsc.subcore_barrier()
        fill(1.0)
        pltpu.make_async_copy(dst_hbm.at[pl.ds(wid * rt, rt)], idx_d,
                              gsem).wait()

        batch = 2 * JB

        def ebody(t, carry):
            sdesc = [pltpu.async_copy(buf, acc.at[idx_d.at[t * batch + j]],
                                      ssem, add=True)
                     for j in range(batch)]
            for dsc in sdesc:
                dsc.wait()
            return carry

        lax.fori_loop(0, rt // batch, ebody, 0)
        plsc.subcore_barrier()

        odesc = []
        for i in range(npt // RZ):
            r0 = s * npt + i * RZ
            odesc.append(pltpu.async_copy(
                acc.at[pl.ds(r0, RZ)], out_hbm.at[c].at[pl.ds(r0, RZ)],
                gsem))
        for dsc in odesc:
            dsc.wait()

    return deg


def _dinv(dp, n):
    deg = dp[0, :n, 0:1] + dp[1, :n, 0:1] + 1.0
    return lax.rsqrt(deg)


def _tc_first(x, w1, dparts):
    n, d_out = x.shape[0], w1.shape[1]

    def body(x_ref, w_ref, dp_ref, h_ref, g_ref):
        dinv = _dinv(dp_ref[...], n)
        h = jnp.dot(x_ref[...], w_ref[...], preferred_element_type=jnp.float32)
        h_ref[...] = h
        g_ref[...] = h * dinv

    return pl.pallas_call(
        body,
        out_shape=(jax.ShapeDtypeStruct((n, d_out), jnp.float32),
                   jax.ShapeDtypeStruct((n, d_out), jnp.float32)),
    )(x, w1, dparts)


def _tc_mid(h_prev, parts, w, b_prev, dparts):
    n, d_out = h_prev.shape[0], w.shape[1]

    def body(h_ref, p_ref, w_ref, b_ref, dp_ref, h2_ref, g2_ref):
        dinv = _dinv(dp_ref[...], n)
        p = p_ref[...]
        agg = p[0, :n] + p[1, :n]
        a = jnp.maximum(dinv * agg + (dinv * dinv) * h_ref[...] + b_ref[...],
                        0.0)
        h2 = jnp.dot(a, w_ref[...], preferred_element_type=jnp.float32)
        h2_ref[...] = h2
        g2_ref[...] = h2 * dinv

    return pl.pallas_call(
        body,
        out_shape=(jax.ShapeDtypeStruct((n, d_out), jnp.float32),
                   jax.ShapeDtypeStruct((n, d_out), jnp.float32)),
    )(h_prev, parts, w, b_prev, dparts)


def _tc_final(h_prev, parts, b_prev, dparts):
    n, d = h_prev.shape

    def body(h_ref, p_ref, b_ref, dp_ref, o_ref):
        dinv = _dinv(dp_ref[...], n)
        p = p_ref[...]
        agg = p[0, :n] + p[1, :n]
        a = jnp.maximum(dinv * agg + (dinv * dinv) * h_ref[...] + b_ref[...],
                        0.0)
        z = a - jnp.max(a, axis=1, keepdims=True)
        lse = jnp.log(jnp.sum(jnp.exp(z), axis=1, keepdims=True))
        o_ref[...] = z - lse

    return pl.pallas_call(
        body,
        out_shape=jax.ShapeDtypeStruct((n, d), jnp.float32),
    )(h_prev, parts, b_prev, dparts)


def kernel(x, edge_index, W1, b1, W2, b2, W3, b3):
    n = x.shape[0]
    e = edge_index.shape[1]
    e_pad, n_acc, _, _ = _pad_sizes(n, e)
    ei = edge_index.astype(jnp.int32)
    npad = e_pad - e
    pad_src = jnp.arange(npad, dtype=jnp.int32) % n
    pad_dst = n + jnp.arange(npad, dtype=jnp.int32) % (n_acc - n)
    src2 = jnp.concatenate([ei[0], pad_src]).reshape(e_pad // EB, EB)
    dst2 = jnp.concatenate([ei[1], pad_dst]).reshape(e_pad // EB, EB)

    dparts = _make_deg(n, e)(dst2)
    h1, g1 = _tc_first(x, W1, dparts)
    p1 = _make_agg(n, e, h1.shape[1])(g1, src2, dst2)
    h2, g2 = _tc_mid(h1, p1, W2, b1.reshape(1, -1), dparts)
    p2 = _make_agg(n, e, h2.shape[1])(g2, src2, dst2)
    h3, g3 = _tc_mid(h2, p2, W3, b2.reshape(1, -1), dparts)
    p3 = _make_agg(n, e, h3.shape[1])(g3, src2, dst2)
    return _tc_final(h3, p3, b3.reshape(1, -1), dparts)


# speedup vs baseline: 2.1881x; 1.0008x over previous
import functools

import jax
import jax.numpy as jnp
from jax import lax
from jax.experimental import pallas as pl
from jax.experimental.pallas import tpu as pltpu
from jax.experimental.pallas import tpu_sc as plsc

NC = 2
NS = 16
NW = NC * NS
EB = 128
JB = 4
BB = 8
DD = 16
RZ = 128


def _pad_sizes(n, e):
    rt = -(-(-(-e // EB) // NW) // (2 * JB)) * (2 * JB)
    e_pad = rt * NW * EB
    npt = -(-(n // NS) // RZ) * RZ
    n_acc = npt * NS
    return e_pad, n_acc, npt, rt


def _make_mesh():
    return plsc.VectorSubcoreMesh(core_axis_name="c", subcore_axis_name="s")


def _make_agg(n, e, d):
    assert n % NS == 0 and d % 16 == 0
    e_pad, n_acc, npt, rt = _pad_sizes(n, e)

    @functools.partial(
        pl.kernel,
        mesh=_make_mesh(),
        out_type=jax.ShapeDtypeStruct((NC, n_acc, d), jnp.float32),
        scratch_types=[
            pltpu.VMEM((rt, EB), jnp.int32),
            pltpu.VMEM((rt, EB), jnp.int32),
            pltpu.VMEM((EB, d), jnp.float32),
            pltpu.VMEM((EB, d), jnp.float32),
            pltpu.VMEM((EB, d), jnp.float32),
            pltpu.VMEM((EB, d), jnp.float32),
            pltpu.VMEM_SHARED((n_acc, d), jnp.float32),
            pltpu.SemaphoreType.DMA,
            pltpu.SemaphoreType.DMA,
            pltpu.SemaphoreType.DMA,
        ],
        compiler_params=pltpu.CompilerParams(use_tc_tiling_on_sc=False),
    )
    def agg(g_hbm, src_hbm, dst_hbm, out_hbm, idx_s, idx_d, rows0, rows1,
            rows2, rows3, acc, gsem, ssem, isem):
        rows = [rows0, rows1, rows2, rows3]
        c = lax.axis_index("c")
        s = lax.axis_index("s")
        wid = s * NC + c

        pltpu.async_copy(src_hbm.at[pl.ds(wid * rt, rt)], idx_s, isem)
        pltpu.async_copy(dst_hbm.at[pl.ds(wid * rt, rt)], idx_d, isem)

        def zrow(i, carry):
            for k in range(d // 16):
                rows0[i, pl.ds(k * 16, 16)] = jnp.zeros((16,), jnp.float32)
            return carry

        lax.fori_loop(0, EB, zrow, 0)
        for i in range(npt // RZ):
            pltpu.sync_copy(rows0, acc.at[pl.ds(s * npt + i * RZ, RZ)])
        pltpu.make_async_copy(src_hbm.at[pl.ds(wid * rt, rt)], idx_s,
                              isem).wait()
        pltpu.make_async_copy(dst_hbm.at[pl.ds(wid * rt, rt)], idx_d,
                              isem).wait()
        plsc.subcore_barrier()

        def ebody(t, carry):
            gd = [None] * BB
            sd = [None] * BB
            gd[0] = pltpu.async_copy(g_hbm.at[idx_s.at[t * BB]],
                                     rows[0], gsem)
            gd[1] = pltpu.async_copy(g_hbm.at[idx_s.at[t * BB + 1]],
                                     rows[1], gsem)
            for j in range(BB):
                gd[j].wait()
                sd[j] = pltpu.async_copy(rows[j % 4],
                                         acc.at[idx_d.at[t * BB + j]],
                                         ssem, add=True)
                if j + 2 < BB:
                    if j - 2 >= 0:
                        sd[j - 2].wait()
                    gd[j + 2] = pltpu.async_copy(
                        g_hbm.at[idx_s.at[t * BB + j + 2]],
                        rows[(j + 2) % 4], gsem)
            for j in range(max(0, BB - 4), BB):
                sd[j].wait()
            return carry

        lax.fori_loop(0, rt // BB, ebody, 0)
        plsc.subcore_barrier()

        for i in range(npt // RZ):
            r0 = s * npt + i * RZ
            pltpu.sync_copy(acc.at[pl.ds(r0, RZ)],
                            out_hbm.at[c].at[pl.ds(r0, RZ)])

    return agg


def _make_deg(n, e):
    assert n % NS == 0
    e_pad, n_acc, npt, rt = _pad_sizes(n, e)

    @functools.partial(
        pl.kernel,
        mesh=_make_mesh(),
        out_type=jax.ShapeDtypeStruct((NC, n_acc, DD), jnp.float32),
        scratch_types=[
            pltpu.VMEM((rt, EB), jnp.int32),
            pltpu.VMEM((EB, DD), jnp.float32),
            pltpu.VMEM_SHARED((n_acc, DD), jnp.float32),
            pltpu.SemaphoreType.DMA,
            pltpu.SemaphoreType.DMA,
        ],
        compiler_params=pltpu.CompilerParams(use_tc_tiling_on_sc=False),
    )
    def deg(dst_hbm, out_hbm, idx_d, buf, acc, gsem, ssem):
        c = lax.axis_index("c")
        s = lax.axis_index("s")
        wid = s * NC + c

        pltpu.async_copy(dst_hbm.at[pl.ds(wid * rt, rt)], idx_d, gsem)

        def fill(v):
            def frow(i, carry):
                buf[i, pl.ds(0, 16)] = jnp.full((16,), v, jnp.float32)
                return carry
            lax.fori_loop(0, EB, frow, 0)

        fill(0.0)
        zdesc = []
        for i in range(npt // RZ):
            zdesc.append(pltpu.async_copy(
                buf.at[pl.ds(0, RZ)],
                acc.at[pl.ds(s * npt + i * RZ, RZ)], ssem))
        for dsc in zdesc:
            dsc.wait()
        plsc.subcore_barrier()
        fill(1.0)
        pltpu.make_async_copy(dst_hbm.at[pl.ds(wid * rt, rt)], idx_d,
                              gsem).wait()

        batch = 2 * JB

        def ebody(t, carry):
            sdesc = [pltpu.async_copy(buf, acc.at[idx_d.at[t * batch + j]],
                                      ssem, add=True)
                     for j in range(batch)]
            for dsc in sdesc:
                dsc.wait()
            return carry

        lax.fori_loop(0, rt // batch, ebody, 0)
        plsc.subcore_barrier()

        odesc = []
        for i in range(npt // RZ):
            r0 = s * npt + i * RZ
            odesc.append(pltpu.async_copy(
                acc.at[pl.ds(r0, RZ)], out_hbm.at[c].at[pl.ds(r0, RZ)],
                gsem))
        for dsc in odesc:
            dsc.wait()

    return deg


def _dinv(dp, n):
    deg = dp[0, :n, 0:1] + dp[1, :n, 0:1] + 1.0
    return lax.rsqrt(deg)


def _tc_first(x, w1, dparts):
    n, d_out = x.shape[0], w1.shape[1]

    def body(x_ref, w_ref, dp_ref, h_ref, g_ref):
        dinv = _dinv(dp_ref[...], n)
        h = jnp.dot(x_ref[...], w_ref[...], preferred_element_type=jnp.float32)
        h_ref[...] = h
        g_ref[...] = h * dinv

    return pl.pallas_call(
        body,
        out_shape=(jax.ShapeDtypeStruct((n, d_out), jnp.float32),
                   jax.ShapeDtypeStruct((n, d_out), jnp.float32)),
    )(x, w1, dparts)


def _tc_mid(h_prev, parts, w, b_prev, dparts):
    n, d_out = h_prev.shape[0], w.shape[1]

    def body(h_ref, p_ref, w_ref, b_ref, dp_ref, h2_ref, g2_ref):
        dinv = _dinv(dp_ref[...], n)
        p = p_ref[...]
        agg = p[0, :n] + p[1, :n]
        a = jnp.maximum(dinv * agg + (dinv * dinv) * h_ref[...] + b_ref[...],
                        0.0)
        h2 = jnp.dot(a, w_ref[...], preferred_element_type=jnp.float32)
        h2_ref[...] = h2
        g2_ref[...] = h2 * dinv

    return pl.pallas_call(
        body,
        out_shape=(jax.ShapeDtypeStruct((n, d_out), jnp.float32),
                   jax.ShapeDtypeStruct((n, d_out), jnp.float32)),
    )(h_prev, parts, w, b_prev, dparts)


def _tc_final(h_prev, parts, b_prev, dparts):
    n, d = h_prev.shape

    def body(h_ref, p_ref, b_ref, dp_ref, o_ref):
        dinv = _dinv(dp_ref[...], n)
        p = p_ref[...]
        agg = p[0, :n] + p[1, :n]
        a = jnp.maximum(dinv * agg + (dinv * dinv) * h_ref[...] + b_ref[...],
                        0.0)
        z = a - jnp.max(a, axis=1, keepdims=True)
        lse = jnp.log(jnp.sum(jnp.exp(z), axis=1, keepdims=True))
        o_ref[...] = z - lse

    return pl.pallas_call(
        body,
        out_shape=jax.ShapeDtypeStruct((n, d), jnp.float32),
    )(h_prev, parts, b_prev, dparts)


def kernel(x, edge_index, W1, b1, W2, b2, W3, b3):
    n = x.shape[0]
    e = edge_index.shape[1]
    e_pad, n_acc, _, _ = _pad_sizes(n, e)
    ei = edge_index.astype(jnp.int32)
    npad = e_pad - e
    pad_src = jnp.arange(npad, dtype=jnp.int32) % n
    pad_dst = n + jnp.arange(npad, dtype=jnp.int32) % (n_acc - n)
    src2 = jnp.concatenate([ei[0], pad_src]).reshape(e_pad // EB, EB)
    dst2 = jnp.concatenate([ei[1], pad_dst]).reshape(e_pad // EB, EB)

    dparts = _make_deg(n, e)(dst2)
    h1, g1 = _tc_first(x, W1, dparts)
    p1 = _make_agg(n, e, h1.shape[1])(g1, src2, dst2)
    h2, g2 = _tc_mid(h1, p1, W2, b1.reshape(1, -1), dparts)
    p2 = _make_agg(n, e, h2.shape[1])(g2, src2, dst2)
    h3, g3 = _tc_mid(h2, p2, W3, b2.reshape(1, -1), dparts)
    p3 = _make_agg(n, e, h3.shape[1])(g3, src2, dst2)
    return _tc_final(h3, p3, b3.reshape(1, -1), dparts)
